# baseline (speedup 1.0000x reference)
import functools

import jax
import jax.numpy as jnp
from jax.experimental import pallas as pl
from jax.experimental.pallas import tpu as pltpu

NUM_EXPERTS = 8
TOP_K = 2
D_MODEL = 768
D_FF = 3072
SEQ = 2048
TM = 128


def _moe_body(x_ref, wr_ref, w1_ref, b1_ref, w2_ref, b2_ref, out_ref, w_scr):
    e = pl.program_id(1)
    x = x_ref[...]

    @pl.when(e == 0)
    def _router():
        logits = jax.lax.dot_general(
            x, wr_ref[...], (((1,), (1,)), ((), ())),
            preferred_element_type=jnp.float32)
        m = jnp.max(logits, axis=1, keepdims=True)
        p = jnp.exp(logits - m)
        p = p / jnp.sum(p, axis=1, keepdims=True)
        iota = jax.lax.broadcasted_iota(jnp.int32, (TM, NUM_EXPERTS), 1)
        m0 = jnp.max(p, axis=1, keepdims=True)
        i0 = jnp.min(jnp.where(p == m0, iota, NUM_EXPERTS), axis=1, keepdims=True)
        p2 = jnp.where(iota == i0, -1.0, p)
        m1 = jnp.max(p2, axis=1, keepdims=True)
        i1 = jnp.min(jnp.where(p2 == m1, iota, NUM_EXPERTS), axis=1, keepdims=True)
        s = m0 + m1 + 1e-8
        w = jnp.where(iota == i0, m0 / s, jnp.where(iota == i1, m1 / s, 0.0))
        w_scr[...] = w

    h = jax.lax.dot_general(
        x, w1_ref[0], (((1,), (1,)), ((), ())),
        preferred_element_type=jnp.float32) + b1_ref[0]
    h = jax.nn.gelu(h)
    y = jax.lax.dot_general(
        h, w2_ref[0], (((1,), (1,)), ((), ())),
        preferred_element_type=jnp.float32) + b2_ref[0]
    lane = jax.lax.broadcasted_iota(jnp.int32, (TM, NUM_EXPERTS), 1)
    wcol = jnp.sum(jnp.where(lane == e, w_scr[...], 0.0), axis=1,
                   keepdims=True)
    contrib = wcol * y

    @pl.when(e == 0)
    def _init():
        out_ref[...] = contrib

    @pl.when(e != 0)
    def _acc():
        out_ref[...] += contrib


def kernel(residual, W_router, W1, b1, W2, b2):
    x = residual.reshape(SEQ, D_MODEL)
    out = pl.pallas_call(
        _moe_body,
        grid=(SEQ // TM, NUM_EXPERTS),
        in_specs=[
            pl.BlockSpec((TM, D_MODEL), lambda i, e: (i, 0)),
            pl.BlockSpec((NUM_EXPERTS, D_MODEL), lambda i, e: (0, 0)),
            pl.BlockSpec((1, D_FF, D_MODEL), lambda i, e: (e, 0, 0)),
            pl.BlockSpec((1, 1, D_FF), lambda i, e: (e, 0, 0)),
            pl.BlockSpec((1, D_MODEL, D_FF), lambda i, e: (e, 0, 0)),
            pl.BlockSpec((1, 1, D_MODEL), lambda i, e: (e, 0, 0)),
        ],
        out_specs=pl.BlockSpec((TM, D_MODEL), lambda i, e: (i, 0)),
        out_shape=jax.ShapeDtypeStruct((SEQ, D_MODEL), jnp.float32),
        scratch_shapes=[pltpu.VMEM((TM, NUM_EXPERTS), jnp.float32)],
        compiler_params=pltpu.CompilerParams(
            dimension_semantics=("arbitrary", "arbitrary")),
    )(x, W_router, W1, b1.reshape(NUM_EXPERTS, 1, D_FF),
      W2, b2.reshape(NUM_EXPERTS, 1, D_MODEL))
    return out.reshape(1, SEQ, D_MODEL)


# speedup vs baseline: 3.9937x; 3.9937x over previous
import functools

import jax
import jax.numpy as jnp
from jax import lax
from jax.experimental import pallas as pl
from jax.experimental.pallas import tpu as pltpu
from jax.experimental.pallas import tpu_sc as plsc

NUM_EXPERTS = 8
TOP_K = 2
D_MODEL = 768
D_FF = 3072
SEQ = 2048

TM = 256
NT = SEQ * TOP_K // TM + NUM_EXPERTS
MPAD = NT * TM

_NC = 2
_NS = 16
_NW = _NC * _NS
CHUNK = SEQ // _NW


def _router_body(x_ref, wr_ref, pos0_ref, pos1_ref, p0_ref, p1_ref, te_ref,
                 rank_scr, a_scr):
    x = x_ref[...]
    logits = lax.dot_general(x, wr_ref[...], (((1,), (1,)), ((), ())),
                             preferred_element_type=jnp.float32)
    m = jnp.max(logits, axis=1, keepdims=True)
    p = jnp.exp(logits - m)
    p = p / jnp.sum(p, axis=1, keepdims=True)

    iota = lax.broadcasted_iota(jnp.int32, (SEQ, NUM_EXPERTS), 1)
    m0 = jnp.max(p, axis=1, keepdims=True)
    i0 = jnp.min(jnp.where(p == m0, iota, NUM_EXPERTS), axis=1, keepdims=True)
    p2 = jnp.where(iota == i0, -1.0, p)
    m1 = jnp.max(p2, axis=1, keepdims=True)
    i1 = jnp.min(jnp.where(p2 == m1, iota, NUM_EXPERTS), axis=1, keepdims=True)
    s = m0 + m1 + 1e-8
    p0_ref[...] = jnp.broadcast_to(m0 / s, (SEQ, 128))
    p1_ref[...] = jnp.broadcast_to(m1 / s, (SEQ, 128))

    a_scr[...] = (jnp.where(iota == i0, 1.0, 0.0) +
                  jnp.where(iota == i1, 1.0, 0.0))

    rt = lax.broadcasted_iota(jnp.int32, (128, 128), 0)
    ct = lax.broadcasted_iota(jnp.int32, (128, 128), 1)
    tril = jnp.where(rt > ct, 1.0, 0.0)

    def blk(i, cum):
        ab = a_scr[pl.ds(i * 128, 128), :]
        local = lax.dot_general(tril, ab, (((1,), (0,)), ((), ())),
                                preferred_element_type=jnp.float32)
        rank_scr[pl.ds(i * 128, 128), :] = local + cum
        return cum + jnp.sum(ab, axis=0, keepdims=True)

    counts = lax.fori_loop(0, SEQ // 128, blk,
                           jnp.zeros((1, NUM_EXPERTS), jnp.float32))

    padded = jnp.floor((counts + (TM - 1)) / TM) * TM
    e_r = lax.broadcasted_iota(jnp.int32, (NUM_EXPERTS, NUM_EXPERTS), 0)
    e_c = lax.broadcasted_iota(jnp.int32, (NUM_EXPERTS, NUM_EXPERTS), 1)
    excl = jnp.where(e_r < e_c, 1.0, 0.0)
    base = lax.dot_general(padded, excl, (((1,), (0,)), ((), ())),
                           preferred_element_type=jnp.float32)

    rank = rank_scr[...]
    tgt = base + rank
    mask0 = jnp.where(iota == i0, 1.0, 0.0)
    mask1 = jnp.where(iota == i1, 1.0, 0.0)
    pos0_ref[...] = jnp.sum(mask0 * tgt, axis=1, keepdims=True).astype(jnp.int32)
    pos1_ref[...] = jnp.sum(mask1 * tgt, axis=1, keepdims=True).astype(jnp.int32)

    ends = base + padded
    tstart = (lax.broadcasted_iota(jnp.int32, (NT, NUM_EXPERTS), 0)
              .astype(jnp.float32) * TM)
    te = jnp.sum(jnp.where(tstart >= ends, 1.0, 0.0), axis=1, keepdims=True)
    te_ref[...] = jnp.minimum(te, NUM_EXPERTS - 1).astype(jnp.int32)


def _router_call(x, W_router):
    return pl.pallas_call(
        _router_body,
        grid=(1,),
        in_specs=[
            pl.BlockSpec((SEQ, D_MODEL), lambda i: (0, 0)),
            pl.BlockSpec((NUM_EXPERTS, D_MODEL), lambda i: (0, 0)),
        ],
        out_specs=[
            pl.BlockSpec((SEQ, 1), lambda i: (0, 0)),
            pl.BlockSpec((SEQ, 1), lambda i: (0, 0)),
            pl.BlockSpec((SEQ, 128), lambda i: (0, 0)),
            pl.BlockSpec((SEQ, 128), lambda i: (0, 0)),
            pl.BlockSpec((NT, 1), lambda i: (0, 0)),
        ],
        out_shape=[
            jax.ShapeDtypeStruct((SEQ, 1), jnp.int32),
            jax.ShapeDtypeStruct((SEQ, 1), jnp.int32),
            jax.ShapeDtypeStruct((SEQ, 128), jnp.float32),
            jax.ShapeDtypeStruct((SEQ, 128), jnp.float32),
            jax.ShapeDtypeStruct((NT, 1), jnp.int32),
        ],
        scratch_shapes=[pltpu.VMEM((SEQ, NUM_EXPERTS), jnp.float32),
                        pltpu.VMEM((SEQ, NUM_EXPERTS), jnp.float32)],
    )(x, W_router)


def _dispatch_body(x_hbm, pos0_hbm, pos1_hbm, xs_hbm,
                   idx0_v, idx1_v, rows_v, sem):
    wid = lax.axis_index("s") * _NC + lax.axis_index("c")
    b = wid * CHUNK
    pltpu.sync_copy(pos0_hbm.at[pl.ds(b, CHUNK)], idx0_v)
    pltpu.sync_copy(pos1_hbm.at[pl.ds(b, CHUNK)], idx1_v)
    pltpu.sync_copy(x_hbm.at[pl.ds(b, CHUNK)], rows_v)
    pltpu.async_copy(rows_v, xs_hbm.at[idx0_v], sem).wait()
    pltpu.async_copy(rows_v, xs_hbm.at[idx1_v], sem).wait()


def _dispatch_call(x, pos0, pos1):
    mesh = plsc.VectorSubcoreMesh(core_axis_name="c", subcore_axis_name="s")
    return pl.kernel(
        _dispatch_body,
        out_type=jax.ShapeDtypeStruct((MPAD, D_MODEL), jnp.float32),
        mesh=mesh,
        scratch_types=[
            pltpu.VMEM((CHUNK,), jnp.int32),
            pltpu.VMEM((CHUNK,), jnp.int32),
            pltpu.VMEM((CHUNK, D_MODEL), jnp.float32),
            pltpu.SemaphoreType.DMA,
        ],
    )(x, pos0, pos1)


def _ffn_body(te_ref, xs_ref, w1_ref, b1_ref, w2_ref, b2_ref, y_ref):
    xt = xs_ref[...]
    h = lax.dot_general(xt, w1_ref[0], (((1,), (1,)), ((), ())),
                        preferred_element_type=jnp.float32) + b1_ref[0]
    h = jax.nn.gelu(h)
    y_ref[...] = lax.dot_general(h, w2_ref[0], (((1,), (1,)), ((), ())),
                                 preferred_element_type=jnp.float32) + b2_ref[0]


def _ffn_call(te, xs, W1, b1, W2, b2):
    grid_spec = pltpu.PrefetchScalarGridSpec(
        num_scalar_prefetch=1,
        grid=(NT,),
        in_specs=[
            pl.BlockSpec((TM, D_MODEL), lambda i, te_r: (i, 0)),
            pl.BlockSpec((1, D_FF, D_MODEL), lambda i, te_r: (te_r[i], 0, 0)),
            pl.BlockSpec((1, 1, D_FF), lambda i, te_r: (te_r[i], 0, 0)),
            pl.BlockSpec((1, D_MODEL, D_FF), lambda i, te_r: (te_r[i], 0, 0)),
            pl.BlockSpec((1, 1, D_MODEL), lambda i, te_r: (te_r[i], 0, 0)),
        ],
        out_specs=pl.BlockSpec((TM, D_MODEL), lambda i, te_r: (i, 0)),
    )
    return pl.pallas_call(
        _ffn_body,
        grid_spec=grid_spec,
        out_shape=jax.ShapeDtypeStruct((MPAD, D_MODEL), jnp.float32),
        compiler_params=pltpu.CompilerParams(
            dimension_semantics=("arbitrary",)),
    )(te, xs, W1, b1.reshape(NUM_EXPERTS, 1, D_FF),
      W2, b2.reshape(NUM_EXPERTS, 1, D_MODEL))


def _combine_body(y_hbm, pos0_hbm, pos1_hbm, p0_hbm, p1_hbm, out_hbm,
                  idx0_v, idx1_v, pvv0, pvv1, rows0_v, rows1_v,
                  sem0, sem1):
    wid = lax.axis_index("s") * _NC + lax.axis_index("c")
    b = wid * CHUNK
    pltpu.sync_copy(pos0_hbm.at[pl.ds(b, CHUNK)], idx0_v)
    pltpu.sync_copy(pos1_hbm.at[pl.ds(b, CHUNK)], idx1_v)
    pltpu.sync_copy(p0_hbm.at[pl.ds(b, CHUNK)], pvv0)
    pltpu.sync_copy(p1_hbm.at[pl.ds(b, CHUNK)], pvv1)
    cp0 = pltpu.async_copy(y_hbm.at[idx0_v], rows0_v, sem0)
    cp1 = pltpu.async_copy(y_hbm.at[idx1_v], rows1_v, sem1)
    cp0.wait()
    cp1.wait()

    for t in range(CHUNK):
        w0 = pvv0[t, pl.ds(0, _NS)]
        w1 = pvv1[t, pl.ds(0, _NS)]

        def col(j, carry, t=t, w0=w0, w1=w1):
            sl = pl.ds(j * _NS, _NS)
            rows0_v[t, sl] = w0 * rows0_v[t, sl] + w1 * rows1_v[t, sl]
            return carry

        lax.fori_loop(0, D_MODEL // _NS, col, 0)
    pltpu.sync_copy(rows0_v, out_hbm.at[pl.ds(b, CHUNK)])


def _combine_call(y, pos0, pos1, p0, p1):
    mesh = plsc.VectorSubcoreMesh(core_axis_name="c", subcore_axis_name="s")
    return pl.kernel(
        _combine_body,
        out_type=jax.ShapeDtypeStruct((SEQ, D_MODEL), jnp.float32),
        mesh=mesh,
        scratch_types=[
            pltpu.VMEM((CHUNK,), jnp.int32),
            pltpu.VMEM((CHUNK,), jnp.int32),
            pltpu.VMEM((CHUNK, 128), jnp.float32),
            pltpu.VMEM((CHUNK, 128), jnp.float32),
            pltpu.VMEM((CHUNK, D_MODEL), jnp.float32),
            pltpu.VMEM((CHUNK, D_MODEL), jnp.float32),
            pltpu.SemaphoreType.DMA,
            pltpu.SemaphoreType.DMA,
        ],
    )(y, pos0, pos1, p0, p1)


def kernel(residual, W_router, W1, b1, W2, b2):
    x = residual.reshape(SEQ, D_MODEL)
    pos0, pos1, p0, p1, te = _router_call(x, W_router)
    pos0 = pos0.reshape(SEQ)
    pos1 = pos1.reshape(SEQ)
    xs = _dispatch_call(x, pos0, pos1)
    y = _ffn_call(te.reshape(NT), xs, W1, b1, W2, b2)
    out = _combine_call(y, pos0, pos1, p0, p1)
    return out.reshape(1, SEQ, D_MODEL)
